# baseline (speedup 1.0000x reference)
import functools

import jax
import jax.numpy as jnp
from jax import lax
from jax.experimental import pallas as pl
from jax.experimental.pallas import tpu as pltpu
from jax.experimental.pallas import tpu_sc as plsc

_VOCAB = 1000000
_HALFV = _VOCAB // 2
_EMBED = 64
_NEG = 10
_BATCH = 16384
_ROWS = _BATCH * (_NEG + 2)
_WIDE = 2 * _EMBED

_NC = 2
_NS = 16
_NW = _NC * _NS
_RPW = _ROWS // _NW
_CHUNK = 128
_NCH = _RPW // _CHUNK
_K = 6
_NGRP = _NCH // _K

_BB = 512
_GRID = _BATCH // _BB
_NBB = _BB * _NEG

_TCOL = 16384


def _transpose_body(xa_ref, out_ref):
    ta = jnp.transpose(xa_ref[...])
    out_ref[...] = jnp.concatenate([ta, ta], axis=1)


def _tc_relayout(embT):
    nblk = (_VOCAB + _TCOL - 1) // _TCOL
    return pl.pallas_call(
        _transpose_body,
        grid_spec=pl.GridSpec(
            grid=(nblk,),
            in_specs=[
                pl.BlockSpec((_EMBED, _TCOL), lambda i: (0, i)),
            ],
            out_specs=pl.BlockSpec((_TCOL, _WIDE), lambda i: (i, 0)),
        ),
        out_shape=jax.ShapeDtypeStruct((_VOCAB, _WIDE), jnp.float32),
        compiler_params=pltpu.CompilerParams(
            vmem_limit_bytes=100 * 1024 * 1024),
    )(embT)


def _sc_gather(table, ids3):
    nch = ids3.shape[1]
    ngrp = nch // _K
    mesh = plsc.VectorSubcoreMesh(
        core_axis_name="c", subcore_axis_name="s",
        num_cores=_NC, num_subcores=_NS)

    @functools.partial(
        pl.kernel,
        out_type=jax.ShapeDtypeStruct((_NW, nch, _CHUNK, _WIDE), jnp.float32),
        mesh=mesh,
        scratch_types=[
            pltpu.VMEM((nch, _CHUNK), jnp.int32),
            pltpu.VMEM((_K, _CHUNK, _WIDE), jnp.float32),
            pltpu.SemaphoreType.DMA,
        ],
        compiler_params=pltpu.CompilerParams(use_tc_tiling_on_sc=True),
    )
    def gather_kernel(emb_hbm, ids_hbm, out_hbm, idx_v, rows_v, sem):
        wid = lax.axis_index("s") * _NC + lax.axis_index("c")
        pltpu.sync_copy(ids_hbm.at[wid], idx_v)
        for g in range(ngrp):
            copies = [
                pltpu.async_copy(
                    emb_hbm.at[idx_v.at[g * _K + j]], rows_v.at[j], sem)
                for j in range(_K)
            ]
            for c in copies:
                c.wait()
            pltpu.sync_copy(rows_v, out_hbm.at[wid, pl.ds(g * _K, _K)])

    return gather_kernel(table, ids3)


def _selu(x):
    alpha = 1.6732632423543772848170429916717
    scale = 1.0507009873554804934193349852946
    return scale * jnp.where(x > 0, x, alpha * (jnp.exp(jnp.minimum(x, 0.0)) - 1.0))


def _log_sigmoid(x):
    return jnp.minimum(x, 0.0) - jnp.log(1.0 + jnp.exp(-jnp.abs(x)))


def _tc_body(neg_ref, c_ref, t_ref, lab_ref,
             encw_ref, encb_ref, decw_ref, decb_ref, deno_ref, cono_ref):
    i = pl.program_id(0)
    w = encw_ref[...]
    b = encb_ref[...]
    enc_c = _selu(jnp.dot(c_ref[:, 0:_EMBED], w,
                          preferred_element_type=jnp.float32) + b)
    enc_t = _selu(jnp.dot(t_ref[:, 0:_EMBED], w,
                          preferred_element_type=jnp.float32) + b)
    enc_n = _selu(jnp.dot(neg_ref[:, 0:_EMBED], w,
                          preferred_element_type=jnp.float32) + b)
    s_true = jnp.sum(enc_c * enc_t, axis=1, keepdims=True)
    n3 = enc_n.reshape(_BB, _NEG, _EMBED)
    s_neg = jnp.sum(n3 * enc_c[:, None, :], axis=2)
    deno_part = (jnp.sum(_log_sigmoid(s_true), keepdims=True)
                 + jnp.sum(_log_sigmoid(-s_neg), keepdims=True))
    logits = jnp.dot(enc_c, decw_ref[...], preferred_element_type=jnp.float32)
    logits = logits + decb_ref[...]
    l0 = logits[:, 0:1]
    l1 = logits[:, 1:2]
    m = jnp.maximum(l0, l1)
    lse = m + jnp.log(jnp.exp(l0 - m) + jnp.exp(l1 - m))
    picked = jnp.where(lab_ref[...] == 0, l0, l1) - lse
    cono_part = jnp.sum(picked, keepdims=True)

    @pl.when(i == 0)
    def _init():
        deno_ref[...] = jnp.zeros((1, 1), jnp.float32)
        cono_ref[...] = jnp.zeros((1, 1), jnp.float32)

    deno_ref[...] += deno_part
    cono_ref[...] += cono_part


def _tc_compute(gathered, labels2, enc_W, enc_b2, dec_W, dec_b2):
    nb = labels2.shape[0]
    c_off = nb * _NEG // _BB
    t_off = nb * (_NEG + 1) // _BB
    grid_spec = pl.GridSpec(
        grid=(nb // _BB,),
        in_specs=[
            pl.BlockSpec((_NBB, _WIDE), lambda i: (i, 0)),
            pl.BlockSpec((_BB, _WIDE), lambda i: (i + c_off, 0)),
            pl.BlockSpec((_BB, _WIDE), lambda i: (i + t_off, 0)),
            pl.BlockSpec((_BB, 1), lambda i: (i, 0)),
            pl.BlockSpec((_EMBED, _EMBED), lambda i: (0, 0)),
            pl.BlockSpec((1, _EMBED), lambda i: (0, 0)),
            pl.BlockSpec((_EMBED, 2), lambda i: (0, 0)),
            pl.BlockSpec((1, 2), lambda i: (0, 0)),
        ],
        out_specs=[
            pl.BlockSpec((1, 1), lambda i: (0, 0)),
            pl.BlockSpec((1, 1), lambda i: (0, 0)),
        ],
    )
    return pl.pallas_call(
        _tc_body,
        grid_spec=grid_spec,
        out_shape=[
            jax.ShapeDtypeStruct((1, 1), jnp.float32),
            jax.ShapeDtypeStruct((1, 1), jnp.float32),
        ],
        compiler_params=pltpu.CompilerParams(
            vmem_limit_bytes=100 * 1024 * 1024),
    )(gathered, gathered, gathered, labels2, enc_W, enc_b2, dec_W, dec_b2)


def kernel(center_word_ids, context_word_ids, party_labels,
           negative_context_ids, embedding, enc_W, enc_b, dec_W, dec_b):
    table = _tc_relayout(jnp.transpose(embedding))
    nslice = 8
    nb = _BATCH // nslice
    deno_sum = jnp.zeros((), jnp.float32)
    cono_sum = jnp.zeros((), jnp.float32)
    for s in range(nslice):
        sl = slice(s * nb, (s + 1) * nb)
        ids = jnp.concatenate([
            negative_context_ids[sl].reshape(-1),
            center_word_ids[sl],
            context_word_ids[sl],
        ]).astype(jnp.int32)
        nrows = nb * (_NEG + 2)
        ids3 = ids.reshape(_NW, nrows // (_NW * _CHUNK), _CHUNK)
        gathered = _sc_gather(table, ids3).reshape(nrows, _WIDE)
        d_s, c_s = _tc_compute(
            gathered,
            party_labels[sl].reshape(nb, 1).astype(jnp.int32),
            enc_W,
            enc_b.reshape(1, _EMBED),
            dec_W,
            dec_b.reshape(1, 2),
        )
        deno_sum += d_s[0, 0]
        cono_sum += c_s[0, 0]
    deno_loss = -(deno_sum / _BATCH)
    cono_loss = -(cono_sum / _BATCH)
    return (deno_loss + cono_loss, deno_loss, cono_loss)


# speedup vs baseline: 1.0605x; 1.0605x over previous
import functools

import jax
import jax.numpy as jnp
from jax import lax
from jax.experimental import pallas as pl
from jax.experimental.pallas import tpu as pltpu
from jax.experimental.pallas import tpu_sc as plsc

_VOCAB = 1000000
_HALFV = _VOCAB // 2
_EMBED = 64
_NEG = 10
_BATCH = 16384
_ROWS = _BATCH * (_NEG + 2)
_WIDE = 2 * _EMBED

_NC = 2
_NS = 16
_NW = _NC * _NS
_RPW = _ROWS // _NW
_CHUNK = 128
_NCH = _RPW // _CHUNK
_K = 6
_NGRP = _NCH // _K

_BB = 512
_GRID = _BATCH // _BB
_NBB = _BB * _NEG

_TCOL = 16384


def _transpose_body(xa_ref, out_ref):
    ta = jnp.transpose(xa_ref[...])
    out_ref[...] = jnp.concatenate([ta, ta], axis=1)


def _tc_relayout(embT):
    nblk = (_VOCAB + _TCOL - 1) // _TCOL
    return pl.pallas_call(
        _transpose_body,
        grid_spec=pl.GridSpec(
            grid=(nblk,),
            in_specs=[
                pl.BlockSpec((_EMBED, _TCOL), lambda i: (0, i)),
            ],
            out_specs=pl.BlockSpec((_TCOL, _WIDE), lambda i: (i, 0)),
        ),
        out_shape=jax.ShapeDtypeStruct((_VOCAB, _WIDE), jnp.float32),
        compiler_params=pltpu.CompilerParams(
            vmem_limit_bytes=100 * 1024 * 1024),
    )(embT)


def _sc_gather(table, ids3):
    nch = ids3.shape[1]
    ngrp = nch // _K
    mesh = plsc.VectorSubcoreMesh(
        core_axis_name="c", subcore_axis_name="s",
        num_cores=_NC, num_subcores=_NS)

    @functools.partial(
        pl.kernel,
        out_type=jax.ShapeDtypeStruct((_NW, nch, _CHUNK, _WIDE), jnp.float32),
        mesh=mesh,
        scratch_types=[
            pltpu.VMEM((nch, _CHUNK), jnp.int32),
            pltpu.VMEM((_K, _CHUNK, _WIDE), jnp.float32),
            pltpu.SemaphoreType.DMA,
        ],
        compiler_params=pltpu.CompilerParams(use_tc_tiling_on_sc=True),
    )
    def gather_kernel(emb_hbm, ids_hbm, out_hbm, idx_v, rows_v, sem):
        wid = lax.axis_index("s") * _NC + lax.axis_index("c")
        pltpu.sync_copy(ids_hbm.at[wid], idx_v)
        for g in range(ngrp):
            copies = [
                pltpu.async_copy(
                    emb_hbm.at[idx_v.at[g * _K + j]], rows_v.at[j], sem)
                for j in range(_K)
            ]
            for c in copies:
                c.wait()
            pltpu.sync_copy(rows_v, out_hbm.at[wid, pl.ds(g * _K, _K)])

    return gather_kernel(table, ids3)


def _selu(x):
    alpha = 1.6732632423543772848170429916717
    scale = 1.0507009873554804934193349852946
    return scale * jnp.where(x > 0, x, alpha * (jnp.exp(jnp.minimum(x, 0.0)) - 1.0))


def _log_sigmoid(x):
    return jnp.minimum(x, 0.0) - jnp.log(1.0 + jnp.exp(-jnp.abs(x)))


def _tc_body(neg_ref, c_ref, t_ref, lab_ref,
             encw_ref, encb_ref, decw_ref, decb_ref, deno_ref, cono_ref):
    i = pl.program_id(0)
    w = encw_ref[...]
    b = encb_ref[...]
    enc_c = _selu(jnp.dot(c_ref[:, 0:_EMBED], w,
                          preferred_element_type=jnp.float32) + b)
    enc_t = _selu(jnp.dot(t_ref[:, 0:_EMBED], w,
                          preferred_element_type=jnp.float32) + b)
    enc_n = _selu(jnp.dot(neg_ref[:, 0:_EMBED], w,
                          preferred_element_type=jnp.float32) + b)
    s_true = jnp.sum(enc_c * enc_t, axis=1, keepdims=True)
    n3 = enc_n.reshape(_BB, _NEG, _EMBED)
    s_neg = jnp.sum(n3 * enc_c[:, None, :], axis=2)
    deno_part = (jnp.sum(_log_sigmoid(s_true), keepdims=True)
                 + jnp.sum(_log_sigmoid(-s_neg), keepdims=True))
    logits = jnp.dot(enc_c, decw_ref[...], preferred_element_type=jnp.float32)
    logits = logits + decb_ref[...]
    l0 = logits[:, 0:1]
    l1 = logits[:, 1:2]
    m = jnp.maximum(l0, l1)
    lse = m + jnp.log(jnp.exp(l0 - m) + jnp.exp(l1 - m))
    picked = jnp.where(lab_ref[...] == 0, l0, l1) - lse
    cono_part = jnp.sum(picked, keepdims=True)

    @pl.when(i == 0)
    def _init():
        deno_ref[...] = jnp.zeros((1, 1), jnp.float32)
        cono_ref[...] = jnp.zeros((1, 1), jnp.float32)

    deno_ref[...] += deno_part
    cono_ref[...] += cono_part


def _tc_compute(gathered, labels2, enc_W, enc_b2, dec_W, dec_b2):
    nb = labels2.shape[0]
    c_off = nb * _NEG // _BB
    t_off = nb * (_NEG + 1) // _BB
    grid_spec = pl.GridSpec(
        grid=(nb // _BB,),
        in_specs=[
            pl.BlockSpec((_NBB, _WIDE), lambda i: (i, 0)),
            pl.BlockSpec((_BB, _WIDE), lambda i: (i + c_off, 0)),
            pl.BlockSpec((_BB, _WIDE), lambda i: (i + t_off, 0)),
            pl.BlockSpec((_BB, 1), lambda i: (i, 0)),
            pl.BlockSpec((_EMBED, _EMBED), lambda i: (0, 0)),
            pl.BlockSpec((1, _EMBED), lambda i: (0, 0)),
            pl.BlockSpec((_EMBED, 2), lambda i: (0, 0)),
            pl.BlockSpec((1, 2), lambda i: (0, 0)),
        ],
        out_specs=[
            pl.BlockSpec((1, 1), lambda i: (0, 0)),
            pl.BlockSpec((1, 1), lambda i: (0, 0)),
        ],
    )
    return pl.pallas_call(
        _tc_body,
        grid_spec=grid_spec,
        out_shape=[
            jax.ShapeDtypeStruct((1, 1), jnp.float32),
            jax.ShapeDtypeStruct((1, 1), jnp.float32),
        ],
        compiler_params=pltpu.CompilerParams(
            vmem_limit_bytes=100 * 1024 * 1024),
    )(gathered, gathered, gathered, labels2, enc_W, enc_b2, dec_W, dec_b2)


def kernel(center_word_ids, context_word_ids, party_labels,
           negative_context_ids, embedding, enc_W, enc_b, dec_W, dec_b):
    table = _tc_relayout(jnp.transpose(embedding))
    nslice = 4
    nb = _BATCH // nslice
    deno_sum = jnp.zeros((), jnp.float32)
    cono_sum = jnp.zeros((), jnp.float32)
    for s in range(nslice):
        sl = slice(s * nb, (s + 1) * nb)
        ids = jnp.concatenate([
            negative_context_ids[sl].reshape(-1),
            center_word_ids[sl],
            context_word_ids[sl],
        ]).astype(jnp.int32)
        nrows = nb * (_NEG + 2)
        ids3 = ids.reshape(_NW, nrows // (_NW * _CHUNK), _CHUNK)
        gathered = _sc_gather(table, ids3).reshape(nrows, _WIDE)
        d_s, c_s = _tc_compute(
            gathered,
            party_labels[sl].reshape(nb, 1).astype(jnp.int32),
            enc_W,
            enc_b.reshape(1, _EMBED),
            dec_W,
            dec_b.reshape(1, 2),
        )
        deno_sum += d_s[0, 0]
        cono_sum += c_s[0, 0]
    deno_loss = -(deno_sum / _BATCH)
    cono_loss = -(cono_sum / _BATCH)
    return (deno_loss + cono_loss, deno_loss, cono_loss)
